# baseline (speedup 1.0000x reference)
import functools

import jax
import jax.numpy as jnp
from jax import lax
from jax.experimental import pallas as pl
from jax.experimental.pallas import tpu as pltpu
from jax.experimental.pallas import tpu_sc as plsc

_NC = 2
_NS = 16
_NW = _NC * _NS
_L = 16
_D = 128
_IDX_CHUNK = 128


def _make_sc_call(B):
    b_per_w = B // _NW
    n_chunks = b_per_w // _IDX_CHUNK
    d_chunks = _D // _L
    mesh = plsc.VectorSubcoreMesh(core_axis_name="c", subcore_axis_name="s",
                                  num_cores=_NC, num_subcores=_NS)

    n_buf = min(4, n_chunks)

    def body(mem_hbm, nodes_hbm, td_hbm, w_hbm, b_hbm, out_hbm,
             idx_v, td_s, w_v, b_v, *bufs_and_sems):
        rows = bufs_and_sems[:n_buf]
        gsems = bufs_and_sems[n_buf:2 * n_buf]
        st_sem = bufs_and_sems[2 * n_buf]
        cid = lax.axis_index("c")
        sid = lax.axis_index("s")
        wid = sid * _NC + cid
        row0 = wid * b_per_w

        pltpu.sync_copy(nodes_hbm.at[pl.ds(row0, b_per_w)], idx_v)
        gathers = [
            pltpu.async_copy(
                mem_hbm.at[idx_v.at[pl.ds(ch * _IDX_CHUNK, _IDX_CHUNK)]],
                rows[ch], gsems[ch])
            for ch in range(n_buf)
        ]
        pltpu.sync_copy(td_hbm.at[pl.ds(row0, b_per_w)],
                        td_s.at[pl.ds(0, b_per_w)])
        pltpu.sync_copy(w_hbm, w_v)
        pltpu.sync_copy(b_hbm, b_v)

        w_ch = [w_v[pl.ds(j * _L, _L)] for j in range(d_chunks)]
        ob_ch = [b_v[pl.ds(j * _L, _L)] + 1.0 for j in range(d_chunks)]

        stores = [None] * n_chunks
        gathers += [None] * (n_chunks - n_buf)
        for ch in range(n_chunks):
            gathers[ch].wait()
            rv = rows[ch % n_buf]

            @plsc.parallel_loop(0, _IDX_CHUNK, 1, unroll=8)
            def row_body(r, rv=rv, ch=ch):
                td_b = td_s[pl.ds(ch * _IDX_CHUNK + r, _L)][0]
                for j in range(d_chunks):
                    sl = pl.ds(j * _L, _L)
                    rv[r, sl] = rv[r, sl] * (td_b * w_ch[j] + ob_ch[j])

            stores[ch] = pltpu.async_copy(
                rv, out_hbm.at[pl.ds(row0 + ch * _IDX_CHUNK, _IDX_CHUNK)],
                st_sem)
            nxt = ch + n_buf
            if nxt < n_chunks:
                stores[ch].wait()
                gathers[nxt] = pltpu.async_copy(
                    mem_hbm.at[idx_v.at[pl.ds(nxt * _IDX_CHUNK, _IDX_CHUNK)]],
                    rv, gsems[ch % n_buf])

        for ch in range(max(0, n_chunks - n_buf), n_chunks):
            stores[ch].wait()

    return functools.partial(
        pl.kernel,
        out_type=jax.ShapeDtypeStruct((B, _D), jnp.float32),
        mesh=mesh,
        scratch_types=(
            [
                pltpu.VMEM((b_per_w,), jnp.int32),
                pltpu.VMEM((b_per_w + _L,), jnp.float32),
                pltpu.VMEM((_D,), jnp.float32),
                pltpu.VMEM((_D,), jnp.float32),
            ]
            + [pltpu.VMEM((_IDX_CHUNK, _D), jnp.float32)
               for _ in range(n_buf)]
            + [pltpu.SemaphoreType.DMA for _ in range(n_buf)]
            + [pltpu.SemaphoreType.DMA]
        ),
    )(body)


@jax.jit
def _run(memory, nodes, time_diffs, W, b):
    B = nodes.shape[0]
    return _make_sc_call(B)(memory, nodes.astype(jnp.int32), time_diffs,
                            W.reshape(-1), b)


def kernel(memory, nodes, time_diffs, W, b):
    return _run(memory, nodes, time_diffs, W, b)


# speedup vs baseline: 1.2727x; 1.2727x over previous
import functools

import jax
import jax.numpy as jnp
from jax import lax
from jax.experimental import pallas as pl
from jax.experimental.pallas import tpu as pltpu
from jax.experimental.pallas import tpu_sc as plsc

_NC = 2
_NS = 16
_NW = _NC * _NS
_L = 16
_D = 128
_IDX_CHUNK = 128


def _make_sc_call(B):
    b_per_w = B // _NW
    n_chunks = b_per_w // _IDX_CHUNK
    d_chunks = _D // _L
    mesh = plsc.VectorSubcoreMesh(core_axis_name="c", subcore_axis_name="s",
                                  num_cores=_NC, num_subcores=_NS)

    n_buf = min(4, n_chunks)

    def body(mem_hbm, nodes_hbm, td_hbm, w_hbm, b_hbm, out_hbm,
             idx_v, td_s, w_v, b_v, *bufs_and_sems):
        rows = bufs_and_sems[:n_buf]
        gsems = bufs_and_sems[n_buf:2 * n_buf]
        st_sem = bufs_and_sems[2 * n_buf]
        cid = lax.axis_index("c")
        sid = lax.axis_index("s")
        wid = sid * _NC + cid
        row0 = wid * b_per_w

        pltpu.sync_copy(nodes_hbm.at[pl.ds(row0, b_per_w)], idx_v)
        gathers = [
            pltpu.async_copy(
                mem_hbm.at[idx_v.at[pl.ds(ch * _IDX_CHUNK, _IDX_CHUNK)]],
                rows[ch], gsems[ch])
            for ch in range(n_buf)
        ]
        pltpu.sync_copy(td_hbm.at[pl.ds(row0, b_per_w)],
                        td_s.at[pl.ds(0, b_per_w)])
        pltpu.sync_copy(w_hbm, w_v)
        pltpu.sync_copy(b_hbm, b_v)

        w_ch = [w_v[pl.ds(j * _L, _L)] for j in range(d_chunks)]
        ob_ch = [b_v[pl.ds(j * _L, _L)] + 1.0 for j in range(d_chunks)]

        stores = [None] * n_chunks
        gathers += [None] * (n_chunks - n_buf)
        for ch in range(n_chunks):
            gathers[ch].wait()
            rv = rows[ch % n_buf]

            @plsc.parallel_loop(0, _IDX_CHUNK, 1, unroll=4)
            def row_body(r, rv=rv, ch=ch):
                td_b = td_s[pl.ds(ch * _IDX_CHUNK + r, _L)][0]
                for j in range(d_chunks):
                    sl = pl.ds(j * _L, _L)
                    rv[r, sl] = rv[r, sl] * (td_b * w_ch[j] + ob_ch[j])

            stores[ch] = pltpu.async_copy(
                rv, out_hbm.at[pl.ds(row0 + ch * _IDX_CHUNK, _IDX_CHUNK)],
                st_sem)
            nxt = ch + n_buf
            if nxt < n_chunks:
                stores[ch].wait()
                gathers[nxt] = pltpu.async_copy(
                    mem_hbm.at[idx_v.at[pl.ds(nxt * _IDX_CHUNK, _IDX_CHUNK)]],
                    rv, gsems[ch % n_buf])

        for ch in range(max(0, n_chunks - n_buf), n_chunks):
            stores[ch].wait()

    return functools.partial(
        pl.kernel,
        out_type=jax.ShapeDtypeStruct((B, _D), jnp.float32),
        mesh=mesh,
        scratch_types=(
            [
                pltpu.VMEM((b_per_w,), jnp.int32),
                pltpu.VMEM((b_per_w + _L,), jnp.float32),
                pltpu.VMEM((_D,), jnp.float32),
                pltpu.VMEM((_D,), jnp.float32),
            ]
            + [pltpu.VMEM((_IDX_CHUNK, _D), jnp.float32)
               for _ in range(n_buf)]
            + [pltpu.SemaphoreType.DMA for _ in range(n_buf)]
            + [pltpu.SemaphoreType.DMA]
        ),
    )(body)


@jax.jit
def _run(memory, nodes, time_diffs, W, b):
    B = nodes.shape[0]
    return _make_sc_call(B)(memory, nodes.astype(jnp.int32), time_diffs,
                            W.reshape(-1), b)


def kernel(memory, nodes, time_diffs, W, b):
    return _run(memory, nodes, time_diffs, W, b)
